# baseline (speedup 1.0000x reference)
import functools

import jax
import jax.numpy as jnp
import numpy as np
from jax import lax
from jax.experimental import pallas as pl
from jax.experimental.pallas import tpu as pltpu
from jax.experimental.pallas import tpu_sc as plsc

R = 32
D = 512
K = 6
M_REG = 8
FB_ALPHA = 0.1
FB_SCALE = 1.0 / np.sqrt(M_REG)

R_SC = 8
R_TC = R - R_SC
RB = 4
NSC_CORES = 2
RPC = R_SC // NSC_CORES
CH = 64
NCHUNK = D // CH
LANES = 16
YROWS = 16


def _prologue_kernel(h_ref, coords_ref, wreg_ref, betas_ref, mask_ref,
                     nbr_col_ref, out_ref):
    lane = lax.broadcasted_iota(jnp.int32, (R * K, R), 1)
    nbr = nbr_col_ref[...]
    oh_nbr = (lane == nbr).astype(jnp.float32)
    own = lax.broadcasted_iota(jnp.int32, (R * K, R), 0) // K
    oh_own = (lane == own).astype(jnp.float32)
    delta = jax.lax.dot_general(
        oh_own - oh_nbr, coords_ref[...],
        (((1,), (0,)), ((), ())), preferred_element_type=jnp.float32)
    S = jax.lax.dot_general(
        delta, wreg_ref[...],
        (((1,), (1,)), ((), ())), preferred_element_type=jnp.float32)
    fb = jnp.cos(S) * betas_ref[0:1, :] + jnp.sin(S) * betas_ref[1:2, :]
    b = jnp.sum(fb, axis=1, keepdims=True)
    maskN = jax.lax.dot_general(
        oh_nbr, mask_ref[...],
        (((1,), (0,)), ((), ())), preferred_element_type=jnp.float32)
    coeff = (1.0 + (FB_ALPHA * FB_SCALE) * b) * maskN
    hs = jax.lax.dot_general(
        oh_nbr, h_ref[...],
        (((1,), (0,)), ((), ())), preferred_element_type=jnp.float32)
    out_ref[...] = hs * coeff


def _combine_kernel(mtc_ref, ys_ref, out_ref):
    out_ref[0:R_TC, :] = mtc_ref[...].reshape(R_TC, D)
    row = lax.broadcasted_iota(jnp.int32, (R_SC, R_SC * K), 0)
    col = lax.broadcasted_iota(jnp.int32, (R_SC, R_SC * K), 1)
    A = (col // K == row).astype(jnp.float32)
    out_ref[R_TC:R, :] = jax.lax.dot_general(
        A, ys_ref[...], (((1,), (0,)), ((), ())),
        preferred_element_type=jnp.float32)


def _tc_main_kernel(hs_ref, w_ref, out_ref):
    g = pl.program_id(0)
    for rb in range(RB):
        r = g * RB + rb
        acc = jnp.zeros((1, D), dtype=jnp.float32)
        for k in range(K):
            h = hs_ref[pl.ds(r * K + k, 1), :]
            y = jax.lax.dot_general(
                h, w_ref[rb, k],
                (((1,), (1,)), ((), ())),
                preferred_element_type=jnp.float32)
            acc = acc + y
        out_ref[rb] = acc


def _sc_edge_gemv(hs_hbm, w3_hbm, e_glob, hsbuf, wbufs, sems, ysmem, ytile):
    pltpu.sync_copy(hs_hbm.at[e_glob], hsbuf)
    hv = [hsbuf[pl.ds(c * LANES, LANES)] for c in range(D // LANES)]
    rowbase = e_glob * D

    def _compute_chunk(ch, wbuf):
        @pl.loop(0, CH)
        def _orow(o):
            vaccs = [wbuf[o, pl.ds(c * LANES, LANES)] * hv[c]
                     for c in range(4)]
            for c in range(4, D // LANES):
                vaccs[c % 4] = vaccs[c % 4] + (
                    wbuf[o, pl.ds(c * LANES, LANES)] * hv[c])
            vacc = (vaccs[0] + vaccs[1]) + (vaccs[2] + vaccs[3])
            ysmem[ch * CH + o] = jnp.sum(vacc)

    def _issue(ch, buf, sem):
        return pltpu.async_copy(
            w3_hbm.at[pl.ds(rowbase + ch * CH, CH)], buf, sem)

    cps = {0: _issue(0, wbufs[0], sems[0]), 1: _issue(1, wbufs[1], sems[1])}
    for ch in range(NCHUNK):
        cps[ch].wait()
        _compute_chunk(ch, wbufs[ch % 2])
        if ch + 2 < NCHUNK:
            cps[ch + 2] = _issue(ch + 2, wbufs[ch % 2], sems[ch % 2])

    lane = lax.iota(jnp.int32, LANES)

    @pl.loop(0, D // LANES)
    def _ogroup(og):
        yv = jnp.zeros((LANES,), dtype=jnp.float32)
        for j in range(LANES):
            yv = jnp.where(lane == j, ysmem[og * LANES + j], yv)
        row = og // 2
        col = (og - row * 2) * LANES
        ytile[row, pl.ds(col, LANES)] = yv


def _sc_router_kernel(hs_hbm, w3_hbm, out_hbm, hsbuf, wbuf0, wbuf1,
                      yt0, yt1, ysmem, sem0, sem1):
    c = lax.axis_index("c")
    s = lax.axis_index("s")
    n_edges = K * RPC
    base_r = R_TC + c * RPC
    wbufs = (wbuf0, wbuf1)
    sems = (sem0, sem1)

    for t, yt in ((0, yt0), (1, yt1)):
        e_core = s + t * 16
        valid = e_core < n_edges
        r_loc = e_core // K
        k_slot = e_core - r_loc * K
        e_glob = (base_r + r_loc) * K + k_slot
        e_sc = c * n_edges + e_core

        @pl.when(valid)
        def _do(e_glob=e_glob, e_sc=e_sc, yt=yt):
            _sc_edge_gemv(hs_hbm, w3_hbm, e_glob, hsbuf, wbufs, sems,
                          ysmem, yt)
            pltpu.sync_copy(yt, out_hbm.at[pl.ds(e_sc * YROWS, YROWS)])


def kernel(H, reg_mask_prev, reg_coords, W_edge, W_reg, beta_cos, beta_sin, neighbors):
    betas = jnp.stack([beta_cos, beta_sin])
    mask_col = reg_mask_prev.astype(jnp.float32).reshape(R, 1)
    nbr_col = neighbors.reshape(R * K, 1)

    hs = pl.pallas_call(
        _prologue_kernel,
        grid=(1,),
        in_specs=[
            pl.BlockSpec((R, D), lambda i: (0, 0)),
            pl.BlockSpec((R, 2), lambda i: (0, 0)),
            pl.BlockSpec((M_REG, 2), lambda i: (0, 0)),
            pl.BlockSpec((2, M_REG), lambda i: (0, 0)),
            pl.BlockSpec((R, 1), lambda i: (0, 0)),
            pl.BlockSpec((R * K, 1), lambda i: (0, 0)),
        ],
        out_specs=pl.BlockSpec((R * K, D), lambda i: (0, 0)),
        out_shape=jax.ShapeDtypeStruct((R * K, D), jnp.float32),
    )(H, reg_coords, W_reg, betas, mask_col, nbr_col)

    m_tc3 = pl.pallas_call(
        _tc_main_kernel,
        grid=(R_TC // RB,),
        in_specs=[
            pl.BlockSpec((R * K, D), lambda g: (0, 0)),
            pl.BlockSpec((RB, K, D, D), lambda g: (g, 0, 0, 0)),
        ],
        out_specs=pl.BlockSpec((RB, 1, D), lambda g: (g, 0, 0)),
        out_shape=jax.ShapeDtypeStruct((R_TC, 1, D), jnp.float32),
        compiler_params=pltpu.CompilerParams(
            dimension_semantics=("arbitrary",),
        ),
    )(hs, W_edge)
    m_tc = m_tc3

    w3 = W_edge.reshape(R * K * D, D)
    mesh = plsc.VectorSubcoreMesh(core_axis_name="c", subcore_axis_name="s")
    sc_call = functools.partial(
        pl.kernel,
        out_type=jax.ShapeDtypeStruct((R_SC * K * YROWS, 32), jnp.float32),
        mesh=mesh,
        scratch_types=[
            pltpu.VMEM((D,), jnp.float32),
            pltpu.VMEM((CH, D), jnp.float32),
            pltpu.VMEM((CH, D), jnp.float32),
            pltpu.VMEM((YROWS, 32), jnp.float32),
            pltpu.VMEM((YROWS, 32), jnp.float32),
            pltpu.SMEM((D,), jnp.float32),
            pltpu.SemaphoreType.DMA,
            pltpu.SemaphoreType.DMA,
        ],
        compiler_params=pltpu.CompilerParams(needs_layout_passes=False),
    )(_sc_router_kernel)
    ys = sc_call(hs, w3).reshape(R_SC * K, D)

    return pl.pallas_call(
        _combine_kernel,
        grid=(1,),
        in_specs=[
            pl.BlockSpec((R_TC, 1, D), lambda i: (0, 0, 0)),
            pl.BlockSpec((R_SC * K, D), lambda i: (0, 0)),
        ],
        out_specs=pl.BlockSpec((R, D), lambda i: (0, 0)),
        out_shape=jax.ShapeDtypeStruct((R, D), jnp.float32),
    )(m_tc, ys)


# speedup vs baseline: 1.0151x; 1.0151x over previous
import functools

import jax
import jax.numpy as jnp
import numpy as np
from jax import lax
from jax.experimental import pallas as pl
from jax.experimental.pallas import tpu as pltpu
from jax.experimental.pallas import tpu_sc as plsc

R = 32
D = 512
K = 6
M_REG = 8
FB_ALPHA = 0.1
FB_SCALE = 1.0 / np.sqrt(M_REG)

R_SC = 4
R_TC = R - R_SC
RB = 4
NSC_CORES = 2
RPC = R_SC // NSC_CORES
CH = 64
NCHUNK = D // CH
LANES = 16
YROWS = 16


def _prologue_kernel(h_ref, coords_ref, wreg_ref, betas_ref, mask_ref,
                     nbr_col_ref, out_ref):
    lane = lax.broadcasted_iota(jnp.int32, (R * K, R), 1)
    nbr = nbr_col_ref[...]
    oh_nbr = (lane == nbr).astype(jnp.float32)
    own = lax.broadcasted_iota(jnp.int32, (R * K, R), 0) // K
    oh_own = (lane == own).astype(jnp.float32)
    delta = jax.lax.dot_general(
        oh_own - oh_nbr, coords_ref[...],
        (((1,), (0,)), ((), ())), preferred_element_type=jnp.float32)
    S = jax.lax.dot_general(
        delta, wreg_ref[...],
        (((1,), (1,)), ((), ())), preferred_element_type=jnp.float32)
    fb = jnp.cos(S) * betas_ref[0:1, :] + jnp.sin(S) * betas_ref[1:2, :]
    b = jnp.sum(fb, axis=1, keepdims=True)
    maskN = jax.lax.dot_general(
        oh_nbr, mask_ref[...],
        (((1,), (0,)), ((), ())), preferred_element_type=jnp.float32)
    coeff = (1.0 + (FB_ALPHA * FB_SCALE) * b) * maskN
    hs = jax.lax.dot_general(
        oh_nbr, h_ref[...],
        (((1,), (0,)), ((), ())), preferred_element_type=jnp.float32)
    out_ref[...] = hs * coeff


def _combine_kernel(mtc_ref, ys_ref, out_ref):
    out_ref[0:R_TC, :] = mtc_ref[...].reshape(R_TC, D)
    row = lax.broadcasted_iota(jnp.int32, (R_SC, R_SC * K), 0)
    col = lax.broadcasted_iota(jnp.int32, (R_SC, R_SC * K), 1)
    A = (col // K == row).astype(jnp.float32)
    out_ref[R_TC:R, :] = jax.lax.dot_general(
        A, ys_ref[...], (((1,), (0,)), ((), ())),
        preferred_element_type=jnp.float32)


def _tc_main_kernel(hs_ref, w_ref, out_ref):
    g = pl.program_id(0)
    for rb in range(RB):
        r = g * RB + rb
        acc = jnp.zeros((1, D), dtype=jnp.float32)
        for k in range(K):
            h = hs_ref[pl.ds(r * K + k, 1), :]
            y = jax.lax.dot_general(
                h, w_ref[rb, k],
                (((1,), (1,)), ((), ())),
                preferred_element_type=jnp.float32)
            acc = acc + y
        out_ref[rb] = acc


def _sc_edge_gemv(hs_hbm, w3_hbm, e_glob, hsbuf, wbufs, sems, ysmem, ytile):
    pltpu.sync_copy(hs_hbm.at[e_glob], hsbuf)
    hv = [hsbuf[pl.ds(c * LANES, LANES)] for c in range(D // LANES)]
    rowbase = e_glob * D

    def _compute_chunk(ch, wbuf):
        @pl.loop(0, CH)
        def _orow(o):
            vaccs = [wbuf[o, pl.ds(c * LANES, LANES)] * hv[c]
                     for c in range(4)]
            for c in range(4, D // LANES):
                vaccs[c % 4] = vaccs[c % 4] + (
                    wbuf[o, pl.ds(c * LANES, LANES)] * hv[c])
            vacc = (vaccs[0] + vaccs[1]) + (vaccs[2] + vaccs[3])
            ysmem[ch * CH + o] = jnp.sum(vacc)

    def _issue(ch, buf, sem):
        return pltpu.async_copy(
            w3_hbm.at[pl.ds(rowbase + ch * CH, CH)], buf, sem)

    cps = {0: _issue(0, wbufs[0], sems[0]), 1: _issue(1, wbufs[1], sems[1])}
    for ch in range(NCHUNK):
        cps[ch].wait()
        _compute_chunk(ch, wbufs[ch % 2])
        if ch + 2 < NCHUNK:
            cps[ch + 2] = _issue(ch + 2, wbufs[ch % 2], sems[ch % 2])

    lane = lax.iota(jnp.int32, LANES)

    @pl.loop(0, D // LANES)
    def _ogroup(og):
        yv = jnp.zeros((LANES,), dtype=jnp.float32)
        for j in range(LANES):
            yv = jnp.where(lane == j, ysmem[og * LANES + j], yv)
        row = og // 2
        col = (og - row * 2) * LANES
        ytile[row, pl.ds(col, LANES)] = yv


def _sc_router_kernel(hs_hbm, w3_hbm, out_hbm, hsbuf, wbuf0, wbuf1,
                      yt0, yt1, ysmem, sem0, sem1):
    c = lax.axis_index("c")
    s = lax.axis_index("s")
    n_edges = K * RPC
    base_r = R_TC + c * RPC
    wbufs = (wbuf0, wbuf1)
    sems = (sem0, sem1)

    for t, yt in ((0, yt0), (1, yt1)):
        e_core = s + t * 16
        valid = e_core < n_edges
        r_loc = e_core // K
        k_slot = e_core - r_loc * K
        e_glob = (base_r + r_loc) * K + k_slot
        e_sc = c * n_edges + e_core

        @pl.when(valid)
        def _do(e_glob=e_glob, e_sc=e_sc, yt=yt):
            _sc_edge_gemv(hs_hbm, w3_hbm, e_glob, hsbuf, wbufs, sems,
                          ysmem, yt)
            pltpu.sync_copy(yt, out_hbm.at[pl.ds(e_sc * YROWS, YROWS)])


def kernel(H, reg_mask_prev, reg_coords, W_edge, W_reg, beta_cos, beta_sin, neighbors):
    betas = jnp.stack([beta_cos, beta_sin])
    mask_col = reg_mask_prev.astype(jnp.float32).reshape(R, 1)
    nbr_col = neighbors.reshape(R * K, 1)

    hs = pl.pallas_call(
        _prologue_kernel,
        grid=(1,),
        in_specs=[
            pl.BlockSpec((R, D), lambda i: (0, 0)),
            pl.BlockSpec((R, 2), lambda i: (0, 0)),
            pl.BlockSpec((M_REG, 2), lambda i: (0, 0)),
            pl.BlockSpec((2, M_REG), lambda i: (0, 0)),
            pl.BlockSpec((R, 1), lambda i: (0, 0)),
            pl.BlockSpec((R * K, 1), lambda i: (0, 0)),
        ],
        out_specs=pl.BlockSpec((R * K, D), lambda i: (0, 0)),
        out_shape=jax.ShapeDtypeStruct((R * K, D), jnp.float32),
    )(H, reg_coords, W_reg, betas, mask_col, nbr_col)

    m_tc3 = pl.pallas_call(
        _tc_main_kernel,
        grid=(R_TC // RB,),
        in_specs=[
            pl.BlockSpec((R * K, D), lambda g: (0, 0)),
            pl.BlockSpec((RB, K, D, D), lambda g: (g, 0, 0, 0)),
        ],
        out_specs=pl.BlockSpec((RB, 1, D), lambda g: (g, 0, 0)),
        out_shape=jax.ShapeDtypeStruct((R_TC, 1, D), jnp.float32),
        compiler_params=pltpu.CompilerParams(
            dimension_semantics=("arbitrary",),
        ),
    )(hs, W_edge)
    m_tc = m_tc3

    w3 = W_edge.reshape(R * K * D, D)
    mesh = plsc.VectorSubcoreMesh(core_axis_name="c", subcore_axis_name="s")
    sc_call = functools.partial(
        pl.kernel,
        out_type=jax.ShapeDtypeStruct((R_SC * K * YROWS, 32), jnp.float32),
        mesh=mesh,
        scratch_types=[
            pltpu.VMEM((D,), jnp.float32),
            pltpu.VMEM((CH, D), jnp.float32),
            pltpu.VMEM((CH, D), jnp.float32),
            pltpu.VMEM((YROWS, 32), jnp.float32),
            pltpu.VMEM((YROWS, 32), jnp.float32),
            pltpu.SMEM((D,), jnp.float32),
            pltpu.SemaphoreType.DMA,
            pltpu.SemaphoreType.DMA,
        ],
        compiler_params=pltpu.CompilerParams(needs_layout_passes=False),
    )(_sc_router_kernel)
    ys = sc_call(hs, w3).reshape(R_SC * K, D)

    return pl.pallas_call(
        _combine_kernel,
        grid=(1,),
        in_specs=[
            pl.BlockSpec((R_TC, 1, D), lambda i: (0, 0, 0)),
            pl.BlockSpec((R_SC * K, D), lambda i: (0, 0)),
        ],
        out_specs=pl.BlockSpec((R, D), lambda i: (0, 0)),
        out_shape=jax.ShapeDtypeStruct((R, D), jnp.float32),
    )(m_tc, ys)


# speedup vs baseline: 1.2813x; 1.2622x over previous
import jax
import jax.numpy as jnp
import numpy as np
from jax.experimental import pallas as pl
from jax.experimental.pallas import tpu as pltpu

R = 32
D = 512
K = 6
M_REG = 8
FB_ALPHA = 0.1
FB_SCALE = 1.0 / np.sqrt(M_REG)


RB = 4


def _router_kernel(nbr_smem, h_ref, w_ref, coords_ref, wreg_ref,
                   betas_ref, mask_ref, nbr_col_ref, out_ref, coeff_ref):
    g = pl.program_id(0)

    @pl.when(g == 0)
    def _compute_coeff():
        lane = jax.lax.broadcasted_iota(jnp.int32, (R * K, R), 1)
        nbr = nbr_col_ref[...]
        oh_nbr = (lane == nbr).astype(jnp.float32)
        own = jax.lax.broadcasted_iota(jnp.int32, (R * K, R), 0) // K
        oh_own = (lane == own).astype(jnp.float32)
        delta = jax.lax.dot_general(
            oh_own - oh_nbr, coords_ref[...],
            (((1,), (0,)), ((), ())), preferred_element_type=jnp.float32,
        )
        S = jax.lax.dot_general(
            delta, wreg_ref[...],
            (((1,), (1,)), ((), ())), preferred_element_type=jnp.float32,
        )
        fb = (jnp.cos(S) * betas_ref[0:1, :]
              + jnp.sin(S) * betas_ref[1:2, :])
        b = jnp.sum(fb, axis=1, keepdims=True)
        maskN = jax.lax.dot_general(
            oh_nbr, mask_ref[...],
            (((1,), (0,)), ((), ())), preferred_element_type=jnp.float32,
        )
        coeff_ref[...] = (1.0 + (FB_ALPHA * FB_SCALE) * b) * maskN

    for rb in range(RB):
        r = g * RB + rb
        acc = jnp.zeros((1, D), dtype=jnp.float32)
        for k in range(K):
            idx = nbr_smem[r, k]
            h = h_ref[pl.ds(idx, 1), :]
            y = jax.lax.dot_general(
                h, w_ref[rb, k],
                (((1,), (1,)), ((), ())),
                preferred_element_type=jnp.float32,
            )
            acc = acc + y * coeff_ref[pl.ds(r * K + k, 1), :]
        out_ref[rb] = acc


def kernel(H, reg_mask_prev, reg_coords, W_edge, W_reg, beta_cos, beta_sin, neighbors):
    betas = jnp.stack([beta_cos, beta_sin])
    mask_col = reg_mask_prev.astype(jnp.float32).reshape(R, 1)
    nbr_col = neighbors.reshape(R * K, 1)

    out = pl.pallas_call(
        _router_kernel,
        grid=(R // RB,),
        in_specs=[
            pl.BlockSpec(memory_space=pltpu.SMEM),
            pl.BlockSpec((R, D), lambda g: (0, 0)),
            pl.BlockSpec((RB, K, D, D), lambda g: (g, 0, 0, 0)),
            pl.BlockSpec((R, 2), lambda g: (0, 0)),
            pl.BlockSpec((M_REG, 2), lambda g: (0, 0)),
            pl.BlockSpec((2, M_REG), lambda g: (0, 0)),
            pl.BlockSpec((R, 1), lambda g: (0, 0)),
            pl.BlockSpec((R * K, 1), lambda g: (0, 0)),
        ],
        out_specs=pl.BlockSpec((RB, 1, D), lambda g: (g, 0, 0)),
        out_shape=jax.ShapeDtypeStruct((R, 1, D), jnp.float32),
        scratch_shapes=[pltpu.VMEM((R * K, 1), jnp.float32)],
        compiler_params=pltpu.CompilerParams(
            dimension_semantics=("arbitrary",),
        ),
    )(neighbors, H, W_edge, reg_coords, W_reg, betas, mask_col, nbr_col)
    return out.reshape(R, D)
